# Initial kernel scaffold; baseline (speedup 1.0000x reference)
import functools

import jax
import jax.numpy as jnp
from jax import lax
from jax.experimental import pallas as pl
from jax.experimental.pallas import tpu as pltpu
from jax.experimental.pallas import tpu_sc as plsc

D = 128
LANES = 16
NC = 2
NS = 16
NW = NC * NS
K = 128
N_PAD = 10240
RPT = N_PAD // NS
ZR = 64


def _mesh():
    return plsc.VectorSubcoreMesh(core_axis_name="c", subcore_axis_name="s")


def _wid():
    cid = lax.axis_index("c")
    sid = lax.axis_index("s")
    return cid, sid, sid * NC + cid


def _deg_body(t_tile, dst_hbm, w_hbm, deg_hbm, dstv, wv, zv, deg_sp):
    cid, sid, wid = _wid()

    def zfill(i, _):
        zv[pl.ds(i * LANES, LANES)] = jnp.zeros((LANES,), jnp.float32)
        return 0

    lax.fori_loop(0, RPT // LANES, zfill, 0)
    pltpu.sync_copy(zv, deg_sp.at[pl.ds(sid * RPT, RPT)])
    plsc.subcore_barrier()

    def edge(g, _):
        off = wid * t_tile + g * K
        pltpu.sync_copy(dst_hbm.at[pl.ds(off, K)], dstv)
        pltpu.sync_copy(w_hbm.at[pl.ds(off, K)], wv)
        pltpu.sync_copy(wv, deg_sp.at[dstv], add=True)
        return 0

    lax.fori_loop(0, t_tile // K, edge, 0)
    plsc.subcore_barrier()
    sl = pl.ds(sid * RPT, RPT)
    pltpu.sync_copy(deg_sp.at[sl], deg_hbm.at[cid, sl])


def _deg_call(dst_all, w_all, t_tile):
    kfn = pl.kernel(
        functools.partial(_deg_body, t_tile),
        out_type=jax.ShapeDtypeStruct((NC, N_PAD), jnp.float32),
        mesh=_mesh(),
        scratch_types=[
            pltpu.VMEM((K,), jnp.int32),
            pltpu.VMEM((K,), jnp.float32),
            pltpu.VMEM((RPT,), jnp.float32),
            pltpu.VMEM_SHARED((N_PAD,), jnp.float32),
        ],
    )
    return kfn(dst_all, w_all)


def _dinv_body(d0_ref, d1_ref, o_ref):
    d = d0_ref[...] + d1_ref[...]
    o_ref[...] = jnp.where(d > 0, lax.rsqrt(jnp.maximum(d, 1e-12)), 0.0)


def _dinv_call(deg_parts):
    d0 = deg_parts[0].reshape(N_PAD // D, D)
    d1 = deg_parts[1].reshape(N_PAD // D, D)
    out = pl.pallas_call(
        _dinv_body,
        out_shape=jax.ShapeDtypeStruct((N_PAD // D, D), jnp.float32),
    )(d0, d1)
    return out.reshape(N_PAD)


def _norm_body(t_tile, src_hbm, dst_hbm, w_hbm, dinv_hbm, x_hbm,
               norm_hbm, src1_hbm, dinv_v, x_v, srcv, dstv, wv, normv, src1v):
    cid, sid, wid = _wid()
    pltpu.sync_copy(dinv_hbm, dinv_v)
    pltpu.sync_copy(x_hbm, x_v)

    def edge(g, _):
        off = wid * t_tile + g * K
        pltpu.sync_copy(src_hbm.at[pl.ds(off, K)], srcv)
        pltpu.sync_copy(dst_hbm.at[pl.ds(off, K)], dstv)
        pltpu.sync_copy(w_hbm.at[pl.ds(off, K)], wv)
        for j in range(K // LANES):
            sl = pl.ds(j * LANES, LANES)
            s16 = srcv[sl]
            d16 = dstv[sl]
            nv = plsc.load_gather(dinv_v, [s16]) * wv[sl]
            normv[sl] = nv * plsc.load_gather(dinv_v, [d16])
            src1v[sl] = plsc.load_gather(x_v, [s16])
        pltpu.sync_copy(normv, norm_hbm.at[pl.ds(off, K)])
        pltpu.sync_copy(src1v, src1_hbm.at[pl.ds(off, K)])
        return 0

    lax.fori_loop(0, t_tile // K, edge, 0)


def _norm_call(src_all, dst_all, w_all, dinv, x_pad, t_tile):
    e_pad = t_tile * NW
    kfn = pl.kernel(
        functools.partial(_norm_body, t_tile),
        out_type=(
            jax.ShapeDtypeStruct((e_pad,), jnp.float32),
            jax.ShapeDtypeStruct((e_pad,), jnp.int32),
        ),
        mesh=_mesh(),
        scratch_types=[
            pltpu.VMEM((N_PAD,), jnp.float32),
            pltpu.VMEM((N_PAD,), jnp.int32),
            pltpu.VMEM((K,), jnp.int32),
            pltpu.VMEM((K,), jnp.int32),
            pltpu.VMEM((K,), jnp.float32),
            pltpu.VMEM((K,), jnp.float32),
            pltpu.VMEM((K,), jnp.int32),
        ],
    )
    return kfn(src_all, dst_all, w_all, dinv, x_pad)


def _layer_body(t_tile, y_hbm, src_hbm, dst_hbm, norm_hbm, out_hbm,
                srcv, dstv, normv, rows, zrow, s_sp, sem):
    cid, sid, wid = _wid()

    def zfill(i, _):
        for j in range(D // LANES):
            zrow[i, pl.ds(j * LANES, LANES)] = jnp.zeros((LANES,), jnp.float32)
        return 0

    lax.fori_loop(0, ZR, zfill, 0)

    def zcopy(i, _):
        pltpu.sync_copy(zrow, s_sp.at[pl.ds(sid * RPT + i * ZR, ZR)])
        return 0

    lax.fori_loop(0, RPT // ZR, zcopy, 0)
    plsc.subcore_barrier()

    def edge(g, _):
        off = wid * t_tile + g * K
        pltpu.sync_copy(src_hbm.at[pl.ds(off, K)], srcv)
        pltpu.sync_copy(dst_hbm.at[pl.ds(off, K)], dstv)
        pltpu.sync_copy(norm_hbm.at[pl.ds(off, K)], normv)
        pltpu.async_copy(y_hbm.at[srcv], rows, sem).wait()

        def scale(r4, _):
            for u in range(4):
                r = r4 * 4 + u
                nb = plsc.load_gather(normv, [jnp.full((LANES,), r, jnp.int32)])
                for j in range(D // LANES):
                    sl = pl.ds(j * LANES, LANES)
                    rows[r, sl] = rows[r, sl] * nb
            return 0

        lax.fori_loop(0, K // 4, scale, 0)
        pltpu.sync_copy(rows, s_sp.at[dstv], add=True)
        return 0

    lax.fori_loop(0, t_tile // K, edge, 0)
    plsc.subcore_barrier()

    def ocopy(i, _):
        sl = pl.ds(sid * RPT + i * ZR, ZR)
        pltpu.sync_copy(s_sp.at[sl], out_hbm.at[cid, sl])
        return 0

    lax.fori_loop(0, RPT // ZR, ocopy, 0)


def _layer_call(y, src_ids, dst_all, norm, t_tile):
    kfn = pl.kernel(
        functools.partial(_layer_body, t_tile),
        out_type=jax.ShapeDtypeStruct((NC, N_PAD, D), jnp.float32),
        mesh=_mesh(),
        scratch_types=[
            pltpu.VMEM((K,), jnp.int32),
            pltpu.VMEM((K,), jnp.int32),
            pltpu.VMEM((K,), jnp.float32),
            pltpu.VMEM((K, D), jnp.float32),
            pltpu.VMEM((ZR, D), jnp.float32),
            pltpu.VMEM_SHARED((N_PAD, D), jnp.float32),
            pltpu.SemaphoreType.DMA,
        ],
    )
    return kfn(y, src_ids, dst_all, norm)


def _mm0_body(h_ref, w_ref, o_ref):
    o_ref[...] = jnp.dot(h_ref[...], w_ref[...],
                         preferred_element_type=jnp.float32)


def _mm_body(s0_ref, s1_ref, b_ref, w_ref, o_ref):
    h = s0_ref[...] + s1_ref[...] + b_ref[...]
    o_ref[...] = jnp.dot(h, w_ref[...], preferred_element_type=jnp.float32)


def _fin_body(s0_ref, s1_ref, b_ref, o_ref):
    o_ref[...] = s0_ref[...] + s1_ref[...] + b_ref[...]


_BLK = 256


def _mm0_call(h, w):
    return pl.pallas_call(
        _mm0_body,
        grid=(N_PAD // _BLK,),
        in_specs=[
            pl.BlockSpec((_BLK, D), lambda i: (i, 0)),
            pl.BlockSpec((D, D), lambda i: (0, 0)),
        ],
        out_specs=pl.BlockSpec((_BLK, D), lambda i: (i, 0)),
        out_shape=jax.ShapeDtypeStruct((N_PAD, D), jnp.float32),
    )(h, w)


def _mm_call(s, b, w):
    return pl.pallas_call(
        _mm_body,
        grid=(N_PAD // _BLK,),
        in_specs=[
            pl.BlockSpec((_BLK, D), lambda i: (i, 0)),
            pl.BlockSpec((_BLK, D), lambda i: (i, 0)),
            pl.BlockSpec((1, D), lambda i: (0, 0)),
            pl.BlockSpec((D, D), lambda i: (0, 0)),
        ],
        out_specs=pl.BlockSpec((_BLK, D), lambda i: (i, 0)),
        out_shape=jax.ShapeDtypeStruct((N_PAD, D), jnp.float32),
    )(s[0], s[1], b.reshape(1, D), w)


def _fin_call(s, b):
    return pl.pallas_call(
        _fin_body,
        grid=(N_PAD // _BLK,),
        in_specs=[
            pl.BlockSpec((_BLK, D), lambda i: (i, 0)),
            pl.BlockSpec((_BLK, D), lambda i: (i, 0)),
            pl.BlockSpec((1, D), lambda i: (0, 0)),
        ],
        out_specs=pl.BlockSpec((_BLK, D), lambda i: (i, 0)),
        out_shape=jax.ShapeDtypeStruct((N_PAD, D), jnp.float32),
    )(s[0], s[1], b.reshape(1, D))


def kernel(x, edge_index, edge_weight, emb, W0, b0, W1, b1, W2, b2):
    n = emb.shape[0]
    e = edge_weight.shape[0]
    e_all = e + n
    t_tile = -(-e_all // (NW * K)) * K
    e_pad = t_tile * NW

    loop_idx = jnp.arange(n, dtype=jnp.int32)
    src_all = jnp.concatenate([edge_index[0].astype(jnp.int32), loop_idx])
    dst_all = jnp.concatenate([edge_index[1].astype(jnp.int32), loop_idx])
    w_all = jnp.concatenate([edge_weight, jnp.ones((n,), jnp.float32)])
    src_all = jnp.pad(src_all, (0, e_pad - e_all))
    dst_all = jnp.pad(dst_all, (0, e_pad - e_all))
    w_all = jnp.pad(w_all, (0, e_pad - e_all))
    x_pad = jnp.pad(x.astype(jnp.int32), (0, N_PAD - n))
    emb_pad = jnp.pad(emb, ((0, N_PAD - n), (0, 0)))

    deg_parts = _deg_call(dst_all, w_all, t_tile)
    dinv = _dinv_call(deg_parts)
    norm, src1 = _norm_call(src_all, dst_all, w_all, dinv, x_pad, t_tile)

    y = _mm0_call(emb_pad, W0)
    s = _layer_call(y, src1, dst_all, norm, t_tile)
    y = _mm_call(s, b0, W1)
    s = _layer_call(y, src_all, dst_all, norm, t_tile)
    y = _mm_call(s, b1, W2)
    s = _layer_call(y, src_all, dst_all, norm, t_tile)
    out = _fin_call(s, b2)
    return out[:n]


# speedup vs baseline: 7.2167x; 7.2167x over previous
import functools

import jax
import jax.numpy as jnp
from jax import lax
from jax.experimental import pallas as pl
from jax.experimental.pallas import tpu as pltpu
from jax.experimental.pallas import tpu_sc as plsc

D = 128
LANES = 16
NC = 2
NS = 16
NW = NC * NS
K = 128
N_PAD = 10240
RPT = N_PAD // NS
ZR = 64


def _mesh():
    return plsc.VectorSubcoreMesh(core_axis_name="c", subcore_axis_name="s")


def _wid():
    cid = lax.axis_index("c")
    sid = lax.axis_index("s")
    return cid, sid, sid * NC + cid


def _deg_body(t_tile, dst_hbm, w_hbm, deg_hbm, dstv, wv, zv, deg_sp):
    cid, sid, wid = _wid()

    def zfill(i, _):
        zv[pl.ds(i * LANES, LANES)] = jnp.zeros((LANES,), jnp.float32)
        return 0

    lax.fori_loop(0, RPT // LANES, zfill, 0)
    pltpu.sync_copy(zv, deg_sp.at[pl.ds(sid * RPT, RPT)])
    plsc.subcore_barrier()

    def edge(g, _):
        off = wid * t_tile + g * K
        pltpu.sync_copy(dst_hbm.at[pl.ds(off, K)], dstv)
        pltpu.sync_copy(w_hbm.at[pl.ds(off, K)], wv)
        pltpu.sync_copy(wv, deg_sp.at[dstv], add=True)
        return 0

    lax.fori_loop(0, t_tile // K, edge, 0)
    plsc.subcore_barrier()
    sl = pl.ds(sid * RPT, RPT)
    pltpu.sync_copy(deg_sp.at[sl], deg_hbm.at[cid, sl])


def _deg_call(dst_all, w_all, t_tile):
    kfn = pl.kernel(
        functools.partial(_deg_body, t_tile),
        out_type=jax.ShapeDtypeStruct((NC, N_PAD), jnp.float32),
        mesh=_mesh(),
        compiler_params=pltpu.CompilerParams(needs_layout_passes=False),
        scratch_types=[
            pltpu.VMEM((K,), jnp.int32),
            pltpu.VMEM((K,), jnp.float32),
            pltpu.VMEM((RPT,), jnp.float32),
            pltpu.VMEM_SHARED((N_PAD,), jnp.float32),
        ],
    )
    return kfn(dst_all, w_all)


def _dinv_body(d0_ref, d1_ref, o_ref):
    d = d0_ref[...] + d1_ref[...]
    o_ref[...] = jnp.where(d > 0, lax.rsqrt(jnp.maximum(d, 1e-12)), 0.0)


def _dinv_call(deg_parts):
    d0 = deg_parts[0].reshape(N_PAD // D, D)
    d1 = deg_parts[1].reshape(N_PAD // D, D)
    out = pl.pallas_call(
        _dinv_body,
        out_shape=jax.ShapeDtypeStruct((N_PAD // D, D), jnp.float32),
    )(d0, d1)
    return out.reshape(N_PAD)


def _norm_body(t_tile, src_hbm, dst_hbm, w_hbm, dinv_hbm, x_hbm,
               norm_hbm, src1_hbm, dinv_v, x_v, srcv, dstv, wv, normv, src1v):
    cid, sid, wid = _wid()
    pltpu.sync_copy(dinv_hbm, dinv_v)
    pltpu.sync_copy(x_hbm, x_v)

    def edge(g, _):
        off = wid * t_tile + g * K
        pltpu.sync_copy(src_hbm.at[pl.ds(off, K)], srcv)
        pltpu.sync_copy(dst_hbm.at[pl.ds(off, K)], dstv)
        pltpu.sync_copy(w_hbm.at[pl.ds(off, K)], wv)
        for j in range(K // LANES):
            sl = pl.ds(j * LANES, LANES)
            s16 = srcv[sl]
            d16 = dstv[sl]
            nv = plsc.load_gather(dinv_v, [s16]) * wv[sl]
            normv[sl] = nv * plsc.load_gather(dinv_v, [d16])
            src1v[sl] = plsc.load_gather(x_v, [s16])
        pltpu.sync_copy(normv, norm_hbm.at[pl.ds(off, K)])
        pltpu.sync_copy(src1v, src1_hbm.at[pl.ds(off, K)])
        return 0

    lax.fori_loop(0, t_tile // K, edge, 0)


def _norm_call(src_all, dst_all, w_all, dinv, x_pad, t_tile):
    e_pad = t_tile * NW
    kfn = pl.kernel(
        functools.partial(_norm_body, t_tile),
        out_type=(
            jax.ShapeDtypeStruct((e_pad,), jnp.float32),
            jax.ShapeDtypeStruct((e_pad,), jnp.int32),
        ),
        mesh=_mesh(),
        compiler_params=pltpu.CompilerParams(needs_layout_passes=False),
        scratch_types=[
            pltpu.VMEM((N_PAD,), jnp.float32),
            pltpu.VMEM((N_PAD,), jnp.int32),
            pltpu.VMEM((K,), jnp.int32),
            pltpu.VMEM((K,), jnp.int32),
            pltpu.VMEM((K,), jnp.float32),
            pltpu.VMEM((K,), jnp.float32),
            pltpu.VMEM((K,), jnp.int32),
        ],
    )
    return kfn(src_all, dst_all, w_all, dinv, x_pad)


def _layer_body(t_tile, y_hbm, src_hbm, dst_hbm, norm_hbm, out_hbm,
                srcv, dstv, normv, rows, zrow, s_sp, sem):
    cid, sid, wid = _wid()

    def zfill(i, _):
        for j in range(D // LANES):
            zrow[i, pl.ds(j * LANES, LANES)] = jnp.zeros((LANES,), jnp.float32)
        return 0

    lax.fori_loop(0, ZR, zfill, 0)

    def zcopy(i, _):
        pltpu.sync_copy(zrow, s_sp.at[pl.ds(sid * RPT + i * ZR, ZR)])
        return 0

    lax.fori_loop(0, RPT // ZR, zcopy, 0)
    plsc.subcore_barrier()

    def edge(g, _):
        off = wid * t_tile + g * K
        pltpu.sync_copy(src_hbm.at[pl.ds(off, K)], srcv)
        pltpu.sync_copy(dst_hbm.at[pl.ds(off, K)], dstv)
        pltpu.sync_copy(norm_hbm.at[pl.ds(off, K)], normv)
        pltpu.async_copy(y_hbm.at[srcv], rows, sem).wait()

        def scale(r4, _):
            for u in range(4):
                r = r4 * 4 + u
                nb = plsc.load_gather(normv, [jnp.full((LANES,), r, jnp.int32)])
                for j in range(D // LANES):
                    sl = pl.ds(j * LANES, LANES)
                    rows[r, sl] = rows[r, sl] * nb
            return 0

        lax.fori_loop(0, K // 4, scale, 0)
        pltpu.sync_copy(rows, s_sp.at[dstv], add=True)
        return 0

    lax.fori_loop(0, t_tile // K, edge, 0)
    plsc.subcore_barrier()

    def ocopy(i, _):
        sl = pl.ds(sid * RPT + i * ZR, ZR)
        pltpu.sync_copy(s_sp.at[sl], out_hbm.at[cid, sl])
        return 0

    lax.fori_loop(0, RPT // ZR, ocopy, 0)


def _layer_call(y, src_ids, dst_all, norm, t_tile):
    kfn = pl.kernel(
        functools.partial(_layer_body, t_tile),
        out_type=jax.ShapeDtypeStruct((NC, N_PAD, D), jnp.float32),
        mesh=_mesh(),
        compiler_params=pltpu.CompilerParams(needs_layout_passes=False),
        scratch_types=[
            pltpu.VMEM((K,), jnp.int32),
            pltpu.VMEM((K,), jnp.int32),
            pltpu.VMEM((K,), jnp.float32),
            pltpu.VMEM((K, D), jnp.float32),
            pltpu.VMEM((ZR, D), jnp.float32),
            pltpu.VMEM_SHARED((N_PAD, D), jnp.float32),
            pltpu.SemaphoreType.DMA,
        ],
    )
    return kfn(y, src_ids, dst_all, norm)


def _mm0_body(h_ref, w_ref, o_ref):
    o_ref[...] = jnp.dot(h_ref[...], w_ref[...],
                         preferred_element_type=jnp.float32)


def _mm_body(s0_ref, s1_ref, b_ref, w_ref, o_ref):
    h = s0_ref[...] + s1_ref[...] + b_ref[...]
    o_ref[...] = jnp.dot(h, w_ref[...], preferred_element_type=jnp.float32)


def _fin_body(s0_ref, s1_ref, b_ref, o_ref):
    o_ref[...] = s0_ref[...] + s1_ref[...] + b_ref[...]


_BLK = 256


def _mm0_call(h, w):
    return pl.pallas_call(
        _mm0_body,
        grid=(N_PAD // _BLK,),
        in_specs=[
            pl.BlockSpec((_BLK, D), lambda i: (i, 0)),
            pl.BlockSpec((D, D), lambda i: (0, 0)),
        ],
        out_specs=pl.BlockSpec((_BLK, D), lambda i: (i, 0)),
        out_shape=jax.ShapeDtypeStruct((N_PAD, D), jnp.float32),
    )(h, w)


def _mm_call(s, b, w):
    return pl.pallas_call(
        _mm_body,
        grid=(N_PAD // _BLK,),
        in_specs=[
            pl.BlockSpec((_BLK, D), lambda i: (i, 0)),
            pl.BlockSpec((_BLK, D), lambda i: (i, 0)),
            pl.BlockSpec((1, D), lambda i: (0, 0)),
            pl.BlockSpec((D, D), lambda i: (0, 0)),
        ],
        out_specs=pl.BlockSpec((_BLK, D), lambda i: (i, 0)),
        out_shape=jax.ShapeDtypeStruct((N_PAD, D), jnp.float32),
    )(s[0], s[1], b.reshape(1, D), w)


def _fin_call(s, b):
    return pl.pallas_call(
        _fin_body,
        grid=(N_PAD // _BLK,),
        in_specs=[
            pl.BlockSpec((_BLK, D), lambda i: (i, 0)),
            pl.BlockSpec((_BLK, D), lambda i: (i, 0)),
            pl.BlockSpec((1, D), lambda i: (0, 0)),
        ],
        out_specs=pl.BlockSpec((_BLK, D), lambda i: (i, 0)),
        out_shape=jax.ShapeDtypeStruct((N_PAD, D), jnp.float32),
    )(s[0], s[1], b.reshape(1, D))


def kernel(x, edge_index, edge_weight, emb, W0, b0, W1, b1, W2, b2):
    n = emb.shape[0]
    e = edge_weight.shape[0]
    e_all = e + n
    t_tile = -(-e_all // (NW * K)) * K
    e_pad = t_tile * NW

    loop_idx = jnp.arange(n, dtype=jnp.int32)
    src_all = jnp.concatenate([edge_index[0].astype(jnp.int32), loop_idx])
    dst_all = jnp.concatenate([edge_index[1].astype(jnp.int32), loop_idx])
    w_all = jnp.concatenate([edge_weight, jnp.ones((n,), jnp.float32)])
    src_all = jnp.pad(src_all, (0, e_pad - e_all))
    dst_all = jnp.pad(dst_all, (0, e_pad - e_all))
    w_all = jnp.pad(w_all, (0, e_pad - e_all))
    x_pad = jnp.pad(x.astype(jnp.int32), (0, N_PAD - n))
    emb_pad = jnp.pad(emb, ((0, N_PAD - n), (0, 0)))

    deg_parts = _deg_call(dst_all, w_all, t_tile)
    dinv = _dinv_call(deg_parts)
    norm, src1 = _norm_call(src_all, dst_all, w_all, dinv, x_pad, t_tile)

    y = _mm0_call(emb_pad, W0)
    s = _layer_call(y, src1, dst_all, norm, t_tile)
    y = _mm_call(s, b0, W1)
    s = _layer_call(y, src_all, dst_all, norm, t_tile)
    y = _mm_call(s, b1, W2)
    s = _layer_call(y, src_all, dst_all, norm, t_tile)
    out = _fin_call(s, b2)
    return out[:n]


# speedup vs baseline: 11.4754x; 1.5901x over previous
import functools

import jax
import jax.numpy as jnp
from jax import lax
from jax.experimental import pallas as pl
from jax.experimental.pallas import tpu as pltpu
from jax.experimental.pallas import tpu_sc as plsc

D = 128
LANES = 16
NC = 2
NS = 16
NW = NC * NS
K = 128
N_PAD = 10240
RPT = N_PAD // NS
ZR = 64


def _mesh():
    return plsc.VectorSubcoreMesh(core_axis_name="c", subcore_axis_name="s")


def _wid():
    cid = lax.axis_index("c")
    sid = lax.axis_index("s")
    return cid, sid, sid * NC + cid


def _full16(v):
    return jnp.full((LANES,), v, jnp.int32)


def _deg_body(t_tile, dst_hbm, w_hbm, deg_hbm, dstv, wv, zv, deg_sp):
    cid, sid, wid = _wid()

    def zfill(i, _):
        zv[pl.ds(i * LANES, LANES)] = jnp.zeros((LANES,), jnp.float32)
        return 0

    lax.fori_loop(0, RPT // LANES, zfill, 0)
    pltpu.sync_copy(zv, deg_sp.at[pl.ds(sid * RPT, RPT)])
    plsc.subcore_barrier()

    def edge(g, _):
        off = wid * t_tile + g * K
        pltpu.sync_copy(dst_hbm.at[pl.ds(off, K)], dstv)
        pltpu.sync_copy(w_hbm.at[pl.ds(off, K)], wv)
        pltpu.sync_copy(wv, deg_sp.at[dstv], add=True)
        return 0

    lax.fori_loop(0, t_tile // K, edge, 0)
    plsc.subcore_barrier()
    sl = pl.ds(sid * RPT, RPT)
    pltpu.sync_copy(deg_sp.at[sl], deg_hbm.at[cid, sl])


def _deg_call(dst_all, w_all, t_tile):
    kfn = pl.kernel(
        functools.partial(_deg_body, t_tile),
        out_type=jax.ShapeDtypeStruct((NC, N_PAD), jnp.float32),
        mesh=_mesh(),
        compiler_params=pltpu.CompilerParams(needs_layout_passes=False),
        scratch_types=[
            pltpu.VMEM((K,), jnp.int32),
            pltpu.VMEM((K,), jnp.float32),
            pltpu.VMEM((RPT,), jnp.float32),
            pltpu.VMEM_SHARED((N_PAD,), jnp.float32),
        ],
    )
    return kfn(dst_all, w_all)


def _dinv_body(d0_ref, d1_ref, o_ref):
    d = d0_ref[...] + d1_ref[...]
    o_ref[...] = jnp.where(d > 0, lax.rsqrt(jnp.maximum(d, 1e-12)), 0.0)


def _dinv_call(deg_parts):
    d0 = deg_parts[0].reshape(N_PAD // D, D)
    d1 = deg_parts[1].reshape(N_PAD // D, D)
    out = pl.pallas_call(
        _dinv_body,
        out_shape=jax.ShapeDtypeStruct((N_PAD // D, D), jnp.float32),
    )(d0, d1)
    return out.reshape(N_PAD)


def _norm_body(t_tile, src_hbm, dst_hbm, w_hbm, dinv_hbm, x_hbm,
               norm_hbm, src1_hbm, dinv_v, x_v, srcs, dsts, ws,
               norms, src1s, lsem):
    cid, sid, wid = _wid()
    sl2 = pl.ds(wid * t_tile, t_tile)
    loads = [
        pltpu.async_copy(src_hbm.at[sl2], srcs, lsem),
        pltpu.async_copy(dst_hbm.at[sl2], dsts, lsem),
        pltpu.async_copy(w_hbm.at[sl2], ws, lsem),
        pltpu.async_copy(dinv_hbm, dinv_v, lsem),
        pltpu.async_copy(x_hbm, x_v, lsem),
    ]
    for l in loads:
        l.wait()

    def chunk(g, _):
        for j in range(K // LANES):
            sl = pl.ds(g * K + j * LANES, LANES)
            s16 = srcs[sl]
            d16 = dsts[sl]
            nv = plsc.load_gather(dinv_v, [s16]) * ws[sl]
            norms[sl] = nv * plsc.load_gather(dinv_v, [d16])
            src1s[sl] = plsc.load_gather(x_v, [s16])
        return 0

    lax.fori_loop(0, t_tile // K, chunk, 0)
    pltpu.async_copy(norms, norm_hbm.at[sl2], lsem).wait()
    pltpu.async_copy(src1s, src1_hbm.at[sl2], lsem).wait()


def _norm_call(src_all, dst_all, w_all, dinv, x_pad, t_tile):
    e_pad = t_tile * NW
    kfn = pl.kernel(
        functools.partial(_norm_body, t_tile),
        out_type=(
            jax.ShapeDtypeStruct((e_pad,), jnp.float32),
            jax.ShapeDtypeStruct((e_pad,), jnp.int32),
        ),
        mesh=_mesh(),
        compiler_params=pltpu.CompilerParams(needs_layout_passes=False),
        scratch_types=[
            pltpu.VMEM((N_PAD,), jnp.float32),
            pltpu.VMEM((N_PAD,), jnp.int32),
            pltpu.VMEM((t_tile,), jnp.int32),
            pltpu.VMEM((t_tile,), jnp.int32),
            pltpu.VMEM((t_tile,), jnp.float32),
            pltpu.VMEM((t_tile,), jnp.float32),
            pltpu.VMEM((t_tile,), jnp.int32),
            pltpu.SemaphoreType.DMA,
        ],
    )
    return kfn(src_all, dst_all, w_all, dinv, x_pad)


def _layer_body(t_tile, y_hbm, src_hbm, dst_hbm, norm_hbm, out_hbm,
                srcb0, srcb1, srcb2, srcb3, dstb0, dstb1, dstb2, dstb3,
                normb0, normb1, normb2, normb3, rows0, rows1, zrow, s_sp,
                isem0, isem1, isem2, isem3, gsem0, gsem1, ssem0, ssem1, zsem):
    cid, sid, wid = _wid()
    g_cnt = t_tile // K
    base = wid * t_tile
    srcb = (srcb0, srcb1, srcb2, srcb3)
    dstb = (dstb0, dstb1, dstb2, dstb3)
    normb = (normb0, normb1, normb2, normb3)
    isem = (isem0, isem1, isem2, isem3)
    rows = (rows0, rows1)
    gsem = (gsem0, gsem1)
    ssem = (ssem0, ssem1)

    def ifire(c, s):
        off = base + c * K
        pltpu.async_copy(src_hbm.at[pl.ds(off, K)], srcb[s], isem[s])
        pltpu.async_copy(dst_hbm.at[pl.ds(off, K)], dstb[s], isem[s])
        pltpu.async_copy(norm_hbm.at[pl.ds(off, K)], normb[s], isem[s])

    def iwait(c, s):
        off = base + c * K
        pltpu.make_async_copy(src_hbm.at[pl.ds(off, K)], srcb[s], isem[s]).wait()
        pltpu.make_async_copy(dst_hbm.at[pl.ds(off, K)], dstb[s], isem[s]).wait()
        pltpu.make_async_copy(norm_hbm.at[pl.ds(off, K)], normb[s], isem[s]).wait()

    def gfire(s, r):
        pltpu.async_copy(y_hbm.at[srcb[s]], rows[r], gsem[r])

    def gwait(s, r):
        pltpu.make_async_copy(y_hbm.at[srcb[s]], rows[r], gsem[r]).wait()

    def scale(r, s):
        buf = rows[r]
        nb_ref = normb[s]

        def srow(r4, _):
            for u in range(4):
                rr = r4 * 4 + u
                nb = plsc.load_gather(nb_ref, [_full16(rr)])
                for j in range(D // LANES):
                    sl = pl.ds(j * LANES, LANES)
                    buf[rr, sl] = buf[rr, sl] * nb
            return 0

        lax.fori_loop(0, K // 4, srow, 0)

    def sfire(r, s):
        return pltpu.async_copy(rows[r], s_sp.at[dstb[s]], ssem[r], add=True)

    for c in range(4):
        ifire(c, c)

    def zfill(i, _):
        for j in range(D // LANES):
            zrow[i, pl.ds(j * LANES, LANES)] = jnp.zeros((LANES,), jnp.float32)
        return 0

    lax.fori_loop(0, ZR, zfill, 0)
    zcopies = [
        pltpu.async_copy(zrow, s_sp.at[pl.ds(sid * RPT + i * ZR, ZR)], zsem)
        for i in range(RPT // ZR)
    ]
    for c in zcopies:
        c.wait()
    plsc.subcore_barrier()

    iwait(0, 0)
    gfire(0, 0)
    iwait(1, 1)
    gfire(1, 1)

    quads = g_cnt // 4

    def quad(q, _):
        c0 = 4 * q
        gwait(0, 0)
        scale(0, 0)
        d0 = sfire(0, 0)
        gwait(1, 1)
        scale(1, 1)
        d1 = sfire(1, 1)
        d0.wait()

        @pl.when(c0 + 4 < g_cnt)
        def _():
            ifire(c0 + 4, 0)

        iwait(c0 + 2, 2)
        gfire(2, 0)
        gwait(2, 0)
        scale(0, 2)
        d2 = sfire(0, 2)
        d1.wait()

        @pl.when(c0 + 5 < g_cnt)
        def _():
            ifire(c0 + 5, 1)

        iwait(c0 + 3, 3)
        gfire(3, 1)
        gwait(3, 1)
        scale(1, 3)
        d3 = sfire(1, 3)
        d2.wait()

        @pl.when(c0 + 6 < g_cnt)
        def _():
            ifire(c0 + 6, 2)

        @pl.when(c0 + 4 < g_cnt)
        def _():
            iwait(c0 + 4, 0)
            gfire(0, 0)

        d3.wait()

        @pl.when(c0 + 7 < g_cnt)
        def _():
            ifire(c0 + 7, 3)

        @pl.when(c0 + 5 < g_cnt)
        def _():
            iwait(c0 + 5, 1)
            gfire(1, 1)

        return 0

    lax.fori_loop(0, quads, quad, 0)

    for c in range(4 * quads, g_cnt):
        s = c % 4
        r = c % 2
        gwait(s, r)
        scale(r, s)
        sfire(r, s).wait()

    plsc.subcore_barrier()
    fcopies = [
        pltpu.async_copy(s_sp.at[pl.ds(sid * RPT + i * ZR, ZR)],
                         out_hbm.at[cid, pl.ds(sid * RPT + i * ZR, ZR)], zsem)
        for i in range(RPT // ZR)
    ]
    for c in fcopies:
        c.wait()


def _layer_call(y, src_ids, dst_all, norm, t_tile):
    kfn = pl.kernel(
        functools.partial(_layer_body, t_tile),
        out_type=jax.ShapeDtypeStruct((NC, N_PAD, D), jnp.float32),
        mesh=_mesh(),
        compiler_params=pltpu.CompilerParams(needs_layout_passes=False),
        scratch_types=(
            [pltpu.VMEM((K,), jnp.int32) for _ in range(4)]
            + [pltpu.VMEM((K,), jnp.int32) for _ in range(4)]
            + [pltpu.VMEM((K,), jnp.float32) for _ in range(4)]
            + [
                pltpu.VMEM((K, D), jnp.float32),
                pltpu.VMEM((K, D), jnp.float32),
                pltpu.VMEM((ZR, D), jnp.float32),
                pltpu.VMEM_SHARED((N_PAD, D), jnp.float32),
            ]
            + [pltpu.SemaphoreType.DMA for _ in range(9)]
        ),
    )
    return kfn(y, src_ids, dst_all, norm)


def _mm0_body(h_ref, w_ref, o_ref):
    o_ref[...] = jnp.dot(h_ref[...], w_ref[...],
                         preferred_element_type=jnp.float32)


def _mm_body(s0_ref, s1_ref, b_ref, w_ref, o_ref):
    h = s0_ref[...] + s1_ref[...] + b_ref[...]
    o_ref[...] = jnp.dot(h, w_ref[...], preferred_element_type=jnp.float32)


def _fin_body(s0_ref, s1_ref, b_ref, o_ref):
    o_ref[...] = s0_ref[...] + s1_ref[...] + b_ref[...]


_BLK = 256


def _mm0_call(h, w):
    return pl.pallas_call(
        _mm0_body,
        grid=(N_PAD // _BLK,),
        in_specs=[
            pl.BlockSpec((_BLK, D), lambda i: (i, 0)),
            pl.BlockSpec((D, D), lambda i: (0, 0)),
        ],
        out_specs=pl.BlockSpec((_BLK, D), lambda i: (i, 0)),
        out_shape=jax.ShapeDtypeStruct((N_PAD, D), jnp.float32),
    )(h, w)


def _mm_call(s, b, w):
    return pl.pallas_call(
        _mm_body,
        grid=(N_PAD // _BLK,),
        in_specs=[
            pl.BlockSpec((_BLK, D), lambda i: (i, 0)),
            pl.BlockSpec((_BLK, D), lambda i: (i, 0)),
            pl.BlockSpec((1, D), lambda i: (0, 0)),
            pl.BlockSpec((D, D), lambda i: (0, 0)),
        ],
        out_specs=pl.BlockSpec((_BLK, D), lambda i: (i, 0)),
        out_shape=jax.ShapeDtypeStruct((N_PAD, D), jnp.float32),
    )(s[0], s[1], b.reshape(1, D), w)


def _fin_call(s, b):
    return pl.pallas_call(
        _fin_body,
        grid=(N_PAD // _BLK,),
        in_specs=[
            pl.BlockSpec((_BLK, D), lambda i: (i, 0)),
            pl.BlockSpec((_BLK, D), lambda i: (i, 0)),
            pl.BlockSpec((1, D), lambda i: (0, 0)),
        ],
        out_specs=pl.BlockSpec((_BLK, D), lambda i: (i, 0)),
        out_shape=jax.ShapeDtypeStruct((N_PAD, D), jnp.float32),
    )(s[0], s[1], b.reshape(1, D))


def kernel(x, edge_index, edge_weight, emb, W0, b0, W1, b1, W2, b2):
    n = emb.shape[0]
    e = edge_weight.shape[0]
    e_all = e + n
    t_tile = -(-e_all // (NW * K)) * K
    e_pad = t_tile * NW

    loop_idx = jnp.arange(n, dtype=jnp.int32)
    src_all = jnp.concatenate([edge_index[0].astype(jnp.int32), loop_idx])
    dst_all = jnp.concatenate([edge_index[1].astype(jnp.int32), loop_idx])
    w_all = jnp.concatenate([edge_weight, jnp.ones((n,), jnp.float32)])
    src_all = jnp.pad(src_all, (0, e_pad - e_all))
    dst_all = jnp.pad(dst_all, (0, e_pad - e_all))
    w_all = jnp.pad(w_all, (0, e_pad - e_all))
    x_pad = jnp.pad(x.astype(jnp.int32), (0, N_PAD - n))
    emb_pad = jnp.pad(emb, ((0, N_PAD - n), (0, 0)))

    deg_parts = _deg_call(dst_all, w_all, t_tile)
    dinv = _dinv_call(deg_parts)
    norm, src1 = _norm_call(src_all, dst_all, w_all, dinv, x_pad, t_tile)

    y = _mm0_call(emb_pad, W0)
    s = _layer_call(y, src1, dst_all, norm, t_tile)
    y = _mm_call(s, b0, W1)
    s = _layer_call(y, src_all, dst_all, norm, t_tile)
    y = _mm_call(s, b1, W2)
    s = _layer_call(y, src_all, dst_all, norm, t_tile)
    out = _fin_call(s, b2)
    return out[:n]


# speedup vs baseline: 12.2474x; 1.0673x over previous
import functools

import jax
import jax.numpy as jnp
from jax import lax
from jax.experimental import pallas as pl
from jax.experimental.pallas import tpu as pltpu
from jax.experimental.pallas import tpu_sc as plsc

D = 128
LANES = 16
NC = 2
NS = 16
NW = NC * NS
K = 128
N_PAD = 10240
RPT = N_PAD // NS
ZR = 64


def _mesh():
    return plsc.VectorSubcoreMesh(core_axis_name="c", subcore_axis_name="s")


def _wid():
    cid = lax.axis_index("c")
    sid = lax.axis_index("s")
    return cid, sid, sid * NC + cid


def _full16(v):
    return jnp.full((LANES,), v, jnp.int32)


def _deg_body(t_tile, dst_hbm, w_hbm, deg_hbm,
              dstb0, dstb1, dstb2, dstb3, wb0, wb1, wb2, wb3, zv, deg_sp,
              isem0, isem1, isem2, isem3, ssem0, ssem1, ssem2, ssem3):
    cid, sid, wid = _wid()
    g_cnt = t_tile // K
    base = wid * t_tile
    dstb = (dstb0, dstb1, dstb2, dstb3)
    wb = (wb0, wb1, wb2, wb3)
    isem = (isem0, isem1, isem2, isem3)
    ssem = (ssem0, ssem1, ssem2, ssem3)

    def ifire(c, s):
        off = base + c * K
        pltpu.async_copy(dst_hbm.at[pl.ds(off, K)], dstb[s], isem[s])
        pltpu.async_copy(w_hbm.at[pl.ds(off, K)], wb[s], isem[s])

    def iwait(c, s):
        off = base + c * K
        pltpu.make_async_copy(dst_hbm.at[pl.ds(off, K)], dstb[s], isem[s]).wait()
        pltpu.make_async_copy(w_hbm.at[pl.ds(off, K)], wb[s], isem[s]).wait()

    def sfire(s):
        pltpu.async_copy(wb[s], deg_sp.at[dstb[s]], ssem[s], add=True)

    def swait(s):
        pltpu.make_async_copy(wb[s], deg_sp.at[dstb[s]], ssem[s]).wait()

    for c in range(min(2, g_cnt)):
        ifire(c, c)

    def zfill(i, _):
        zv[pl.ds(i * LANES, LANES)] = jnp.zeros((LANES,), jnp.float32)
        return 0

    lax.fori_loop(0, RPT // LANES, zfill, 0)
    pltpu.sync_copy(zv, deg_sp.at[pl.ds(sid * RPT, RPT)])
    plsc.subcore_barrier()

    quads = g_cnt // 4

    def quad(q, _):
        for j in range(4):
            c = 4 * q + j
            s = j
            s2 = (j + 2) % 4
            iwait(c, s)
            sfire(s)

            @pl.when(c >= 2)
            def _():
                swait(s2)

            @pl.when(c + 2 < g_cnt)
            def _():
                ifire(c + 2, s2)

        return 0

    lax.fori_loop(0, quads, quad, 0)
    for c in range(4 * quads, g_cnt):
        s = c % 4
        s2 = (s + 2) % 4
        iwait(c, s)
        sfire(s)
        if c >= 2:
            swait(s2)
        if c + 2 < g_cnt:
            ifire(c + 2, s2)
    if g_cnt >= 2:
        swait((g_cnt - 2) % 4)
    swait((g_cnt - 1) % 4)
    plsc.subcore_barrier()
    sl = pl.ds(sid * RPT, RPT)
    pltpu.sync_copy(deg_sp.at[sl], deg_hbm.at[cid, sl])


def _deg_call(dst_all, w_all, t_tile):
    kfn = pl.kernel(
        functools.partial(_deg_body, t_tile),
        out_type=jax.ShapeDtypeStruct((NC, N_PAD), jnp.float32),
        mesh=_mesh(),
        compiler_params=pltpu.CompilerParams(needs_layout_passes=False),
        scratch_types=(
            [pltpu.VMEM((K,), jnp.int32) for _ in range(4)]
            + [pltpu.VMEM((K,), jnp.float32) for _ in range(4)]
            + [
                pltpu.VMEM((RPT,), jnp.float32),
                pltpu.VMEM_SHARED((N_PAD,), jnp.float32),
            ]
            + [pltpu.SemaphoreType.DMA for _ in range(8)]
        ),
    )
    return kfn(dst_all, w_all)


def _dinv_body(d0_ref, d1_ref, o_ref):
    d = d0_ref[...] + d1_ref[...]
    o_ref[...] = jnp.where(d > 0, lax.rsqrt(jnp.maximum(d, 1e-12)), 0.0)


def _dinv_call(deg_parts):
    d0 = deg_parts[0].reshape(N_PAD // D, D)
    d1 = deg_parts[1].reshape(N_PAD // D, D)
    out = pl.pallas_call(
        _dinv_body,
        out_shape=jax.ShapeDtypeStruct((N_PAD // D, D), jnp.float32),
    )(d0, d1)
    return out.reshape(N_PAD)


def _norm_body(t_tile, src_hbm, dst_hbm, w_hbm, dinv_hbm, x_hbm,
               norm_hbm, src1_hbm, dinv_v, x_v, srcs, dsts, ws,
               norms, src1s, lsem):
    cid, sid, wid = _wid()
    sl2 = pl.ds(wid * t_tile, t_tile)
    loads = [
        pltpu.async_copy(src_hbm.at[sl2], srcs, lsem),
        pltpu.async_copy(dst_hbm.at[sl2], dsts, lsem),
        pltpu.async_copy(w_hbm.at[sl2], ws, lsem),
        pltpu.async_copy(dinv_hbm, dinv_v, lsem),
        pltpu.async_copy(x_hbm, x_v, lsem),
    ]
    for l in loads:
        l.wait()

    def chunk(g, _):
        for j in range(K // LANES):
            sl = pl.ds(g * K + j * LANES, LANES)
            s16 = srcs[sl]
            d16 = dsts[sl]
            nv = plsc.load_gather(dinv_v, [s16]) * ws[sl]
            norms[sl] = nv * plsc.load_gather(dinv_v, [d16])
            src1s[sl] = plsc.load_gather(x_v, [s16])
        return 0

    lax.fori_loop(0, t_tile // K, chunk, 0)
    pltpu.async_copy(norms, norm_hbm.at[sl2], lsem).wait()
    pltpu.async_copy(src1s, src1_hbm.at[sl2], lsem).wait()


def _norm_call(src_all, dst_all, w_all, dinv, x_pad, t_tile):
    e_pad = t_tile * NW
    kfn = pl.kernel(
        functools.partial(_norm_body, t_tile),
        out_type=(
            jax.ShapeDtypeStruct((e_pad,), jnp.float32),
            jax.ShapeDtypeStruct((e_pad,), jnp.int32),
        ),
        mesh=_mesh(),
        compiler_params=pltpu.CompilerParams(needs_layout_passes=False),
        scratch_types=[
            pltpu.VMEM((N_PAD,), jnp.float32),
            pltpu.VMEM((N_PAD,), jnp.int32),
            pltpu.VMEM((t_tile,), jnp.int32),
            pltpu.VMEM((t_tile,), jnp.int32),
            pltpu.VMEM((t_tile,), jnp.float32),
            pltpu.VMEM((t_tile,), jnp.float32),
            pltpu.VMEM((t_tile,), jnp.int32),
            pltpu.SemaphoreType.DMA,
        ],
    )
    return kfn(src_all, dst_all, w_all, dinv, x_pad)


def _layer_body(t_tile, y_hbm, src_hbm, dst_hbm, norm_hbm, out_hbm,
                srcb0, srcb1, srcb2, srcb3, dstb0, dstb1, dstb2, dstb3,
                normb0, normb1, normb2, normb3, rows0, rows1, zrow, s_sp,
                isem0, isem1, isem2, isem3, gsem0, gsem1, ssem0, ssem1, zsem):
    cid, sid, wid = _wid()
    g_cnt = t_tile // K
    base = wid * t_tile
    srcb = (srcb0, srcb1, srcb2, srcb3)
    dstb = (dstb0, dstb1, dstb2, dstb3)
    normb = (normb0, normb1, normb2, normb3)
    isem = (isem0, isem1, isem2, isem3)
    rows = (rows0, rows1)
    gsem = (gsem0, gsem1)
    ssem = (ssem0, ssem1)

    def ifire(c, s):
        off = base + c * K
        pltpu.async_copy(src_hbm.at[pl.ds(off, K)], srcb[s], isem[s])
        pltpu.async_copy(dst_hbm.at[pl.ds(off, K)], dstb[s], isem[s])
        pltpu.async_copy(norm_hbm.at[pl.ds(off, K)], normb[s], isem[s])

    def iwait(c, s):
        off = base + c * K
        pltpu.make_async_copy(src_hbm.at[pl.ds(off, K)], srcb[s], isem[s]).wait()
        pltpu.make_async_copy(dst_hbm.at[pl.ds(off, K)], dstb[s], isem[s]).wait()
        pltpu.make_async_copy(norm_hbm.at[pl.ds(off, K)], normb[s], isem[s]).wait()

    def gfire(s, r):
        pltpu.async_copy(y_hbm.at[srcb[s]], rows[r], gsem[r])

    def gwait(s, r):
        pltpu.make_async_copy(y_hbm.at[srcb[s]], rows[r], gsem[r]).wait()

    def scale(r, s):
        buf = rows[r]
        nb_ref = normb[s]

        def srow(r4, _):
            for u in range(4):
                rr = r4 * 4 + u
                nb = plsc.load_gather(nb_ref, [_full16(rr)])
                for j in range(D // LANES):
                    sl = pl.ds(j * LANES, LANES)
                    buf[rr, sl] = buf[rr, sl] * nb
            return 0

        lax.fori_loop(0, K // 4, srow, 0)

    def sfire(r, s):
        return pltpu.async_copy(rows[r], s_sp.at[dstb[s]], ssem[r], add=True)

    for c in range(4):
        ifire(c, c)

    def zfill(i, _):
        for j in range(D // LANES):
            zrow[i, pl.ds(j * LANES, LANES)] = jnp.zeros((LANES,), jnp.float32)
        return 0

    lax.fori_loop(0, ZR, zfill, 0)
    zcopies = [
        pltpu.async_copy(zrow, s_sp.at[pl.ds(sid * RPT + i * ZR, ZR)], zsem)
        for i in range(RPT // ZR)
    ]
    iwait(0, 0)
    gfire(0, 0)
    iwait(1, 1)
    gfire(1, 1)
    for c in zcopies:
        c.wait()
    plsc.subcore_barrier()

    quads = g_cnt // 4

    def quad(q, _):
        c0 = 4 * q
        gwait(0, 0)
        scale(0, 0)
        d0 = sfire(0, 0)
        gwait(1, 1)
        scale(1, 1)
        d1 = sfire(1, 1)
        d0.wait()

        @pl.when(c0 + 4 < g_cnt)
        def _():
            ifire(c0 + 4, 0)

        iwait(c0 + 2, 2)
        gfire(2, 0)
        gwait(2, 0)
        scale(0, 2)
        d2 = sfire(0, 2)
        d1.wait()

        @pl.when(c0 + 5 < g_cnt)
        def _():
            ifire(c0 + 5, 1)

        iwait(c0 + 3, 3)
        gfire(3, 1)
        gwait(3, 1)
        scale(1, 3)
        d3 = sfire(1, 3)
        d2.wait()

        @pl.when(c0 + 6 < g_cnt)
        def _():
            ifire(c0 + 6, 2)

        @pl.when(c0 + 4 < g_cnt)
        def _():
            iwait(c0 + 4, 0)
            gfire(0, 0)

        d3.wait()

        @pl.when(c0 + 7 < g_cnt)
        def _():
            ifire(c0 + 7, 3)

        @pl.when(c0 + 5 < g_cnt)
        def _():
            iwait(c0 + 5, 1)
            gfire(1, 1)

        return 0

    lax.fori_loop(0, quads, quad, 0)

    for c in range(4 * quads, g_cnt):
        s = c % 4
        r = c % 2
        gwait(s, r)
        scale(r, s)
        sfire(r, s).wait()

    plsc.subcore_barrier()
    fcopies = [
        pltpu.async_copy(s_sp.at[pl.ds(sid * RPT + i * ZR, ZR)],
                         out_hbm.at[cid, pl.ds(sid * RPT + i * ZR, ZR)], zsem)
        for i in range(RPT // ZR)
    ]
    for c in fcopies:
        c.wait()


def _layer_call(y, src_ids, dst_all, norm, t_tile):
    kfn = pl.kernel(
        functools.partial(_layer_body, t_tile),
        out_type=jax.ShapeDtypeStruct((NC, N_PAD, D), jnp.float32),
        mesh=_mesh(),
        compiler_params=pltpu.CompilerParams(needs_layout_passes=False),
        scratch_types=(
            [pltpu.VMEM((K,), jnp.int32) for _ in range(4)]
            + [pltpu.VMEM((K,), jnp.int32) for _ in range(4)]
            + [pltpu.VMEM((K,), jnp.float32) for _ in range(4)]
            + [
                pltpu.VMEM((K, D), jnp.float32),
                pltpu.VMEM((K, D), jnp.float32),
                pltpu.VMEM((ZR, D), jnp.float32),
                pltpu.VMEM_SHARED((N_PAD, D), jnp.float32),
            ]
            + [pltpu.SemaphoreType.DMA for _ in range(9)]
        ),
    )
    return kfn(y, src_ids, dst_all, norm)


def _mm0_body(h_ref, w_ref, o_ref):
    o_ref[...] = jnp.dot(h_ref[...], w_ref[...],
                         preferred_element_type=jnp.float32)


def _mm_body(s0_ref, s1_ref, b_ref, w_ref, o_ref):
    h = s0_ref[...] + s1_ref[...] + b_ref[...]
    o_ref[...] = jnp.dot(h, w_ref[...], preferred_element_type=jnp.float32)


def _fin_body(s0_ref, s1_ref, b_ref, o_ref):
    o_ref[...] = s0_ref[...] + s1_ref[...] + b_ref[...]


_BLK = 256


def _mm0_call(h, w):
    return pl.pallas_call(
        _mm0_body,
        grid=(N_PAD // _BLK,),
        in_specs=[
            pl.BlockSpec((_BLK, D), lambda i: (i, 0)),
            pl.BlockSpec((D, D), lambda i: (0, 0)),
        ],
        out_specs=pl.BlockSpec((_BLK, D), lambda i: (i, 0)),
        out_shape=jax.ShapeDtypeStruct((N_PAD, D), jnp.float32),
    )(h, w)


def _mm_call(s, b, w):
    return pl.pallas_call(
        _mm_body,
        grid=(N_PAD // _BLK,),
        in_specs=[
            pl.BlockSpec((_BLK, D), lambda i: (i, 0)),
            pl.BlockSpec((_BLK, D), lambda i: (i, 0)),
            pl.BlockSpec((1, D), lambda i: (0, 0)),
            pl.BlockSpec((D, D), lambda i: (0, 0)),
        ],
        out_specs=pl.BlockSpec((_BLK, D), lambda i: (i, 0)),
        out_shape=jax.ShapeDtypeStruct((N_PAD, D), jnp.float32),
    )(s[0], s[1], b.reshape(1, D), w)


def _fin_call(s, b):
    return pl.pallas_call(
        _fin_body,
        grid=(N_PAD // _BLK,),
        in_specs=[
            pl.BlockSpec((_BLK, D), lambda i: (i, 0)),
            pl.BlockSpec((_BLK, D), lambda i: (i, 0)),
            pl.BlockSpec((1, D), lambda i: (0, 0)),
        ],
        out_specs=pl.BlockSpec((_BLK, D), lambda i: (i, 0)),
        out_shape=jax.ShapeDtypeStruct((N_PAD, D), jnp.float32),
    )(s[0], s[1], b.reshape(1, D))


def kernel(x, edge_index, edge_weight, emb, W0, b0, W1, b1, W2, b2):
    n = emb.shape[0]
    e = edge_weight.shape[0]
    e_all = e + n
    t_tile = -(-e_all // (NW * K)) * K
    e_pad = t_tile * NW

    loop_idx = jnp.arange(n, dtype=jnp.int32)
    src_all = jnp.concatenate([edge_index[0].astype(jnp.int32), loop_idx])
    dst_all = jnp.concatenate([edge_index[1].astype(jnp.int32), loop_idx])
    w_all = jnp.concatenate([edge_weight, jnp.ones((n,), jnp.float32)])
    src_all = jnp.pad(src_all, (0, e_pad - e_all))
    dst_all = jnp.pad(dst_all, (0, e_pad - e_all))
    w_all = jnp.pad(w_all, (0, e_pad - e_all))
    x_pad = jnp.pad(x.astype(jnp.int32), (0, N_PAD - n))
    emb_pad = jnp.pad(emb, ((0, N_PAD - n), (0, 0)))

    deg_parts = _deg_call(dst_all, w_all, t_tile)
    dinv = _dinv_call(deg_parts)
    norm, src1 = _norm_call(src_all, dst_all, w_all, dinv, x_pad, t_tile)

    y = _mm0_call(emb_pad, W0)
    s = _layer_call(y, src1, dst_all, norm, t_tile)
    y = _mm_call(s, b0, W1)
    s = _layer_call(y, src_all, dst_all, norm, t_tile)
    y = _mm_call(s, b1, W2)
    s = _layer_call(y, src_all, dst_all, norm, t_tile)
    out = _fin_call(s, b2)
    return out[:n]


# speedup vs baseline: 14.6112x; 1.1930x over previous
import functools

import jax
import jax.numpy as jnp
from jax import lax
from jax.experimental import pallas as pl
from jax.experimental.pallas import tpu as pltpu
from jax.experimental.pallas import tpu_sc as plsc

D = 128
LANES = 16
NC = 2
NS = 16
NW = NC * NS
K = 128
N_PAD = 10240
RPT = N_PAD // NS
ZR = 64


def _mesh():
    return plsc.VectorSubcoreMesh(core_axis_name="c", subcore_axis_name="s")


def _wid():
    cid = lax.axis_index("c")
    sid = lax.axis_index("s")
    return cid, sid, sid * NC + cid


def _full16(v):
    return jnp.full((LANES,), v, jnp.int32)


def _deg_body(t_tile, dst_hbm, w_hbm, deg_hbm,
              dstb0, dstb1, dstb2, dstb3, wb0, wb1, wb2, wb3, zv, deg_sp,
              isem0, isem1, isem2, isem3, ssem0, ssem1, ssem2, ssem3):
    cid, sid, wid = _wid()
    g_cnt = t_tile // K
    base = wid * t_tile
    dstb = (dstb0, dstb1, dstb2, dstb3)
    wb = (wb0, wb1, wb2, wb3)
    isem = (isem0, isem1, isem2, isem3)
    ssem = (ssem0, ssem1, ssem2, ssem3)

    def ifire(c, s):
        off = base + c * K
        pltpu.async_copy(dst_hbm.at[pl.ds(off, K)], dstb[s], isem[s])
        pltpu.async_copy(w_hbm.at[pl.ds(off, K)], wb[s], isem[s])

    def iwait(c, s):
        off = base + c * K
        pltpu.make_async_copy(dst_hbm.at[pl.ds(off, K)], dstb[s], isem[s]).wait()
        pltpu.make_async_copy(w_hbm.at[pl.ds(off, K)], wb[s], isem[s]).wait()

    def sfire(s):
        pltpu.async_copy(wb[s], deg_sp.at[dstb[s]], ssem[s], add=True)

    def swait(s):
        pltpu.make_async_copy(wb[s], deg_sp.at[dstb[s]], ssem[s]).wait()

    for c in range(min(2, g_cnt)):
        ifire(c, c)

    def zfill(i, _):
        zv[pl.ds(i * LANES, LANES)] = jnp.zeros((LANES,), jnp.float32)
        return 0

    lax.fori_loop(0, RPT // LANES, zfill, 0)
    pltpu.sync_copy(zv, deg_sp.at[pl.ds(sid * RPT, RPT)])
    plsc.subcore_barrier()

    quads = g_cnt // 4

    def quad(q, _):
        for j in range(4):
            c = 4 * q + j
            s = j
            s2 = (j + 2) % 4
            iwait(c, s)
            sfire(s)

            @pl.when(c >= 2)
            def _():
                swait(s2)

            @pl.when(c + 2 < g_cnt)
            def _():
                ifire(c + 2, s2)

        return 0

    lax.fori_loop(0, quads, quad, 0)
    for c in range(4 * quads, g_cnt):
        s = c % 4
        s2 = (s + 2) % 4
        iwait(c, s)
        sfire(s)
        if c >= 2:
            swait(s2)
        if c + 2 < g_cnt:
            ifire(c + 2, s2)
    if g_cnt >= 2:
        swait((g_cnt - 2) % 4)
    swait((g_cnt - 1) % 4)
    plsc.subcore_barrier()
    sl = pl.ds(sid * RPT, RPT)
    pltpu.sync_copy(deg_sp.at[sl], deg_hbm.at[cid, sl])


def _deg_call(dst_all, w_all, t_tile):
    kfn = pl.kernel(
        functools.partial(_deg_body, t_tile),
        out_type=jax.ShapeDtypeStruct((NC, N_PAD), jnp.float32),
        mesh=_mesh(),
        compiler_params=pltpu.CompilerParams(needs_layout_passes=False),
        scratch_types=(
            [pltpu.VMEM((K,), jnp.int32) for _ in range(4)]
            + [pltpu.VMEM((K,), jnp.float32) for _ in range(4)]
            + [
                pltpu.VMEM((RPT,), jnp.float32),
                pltpu.VMEM_SHARED((N_PAD,), jnp.float32),
            ]
            + [pltpu.SemaphoreType.DMA for _ in range(8)]
        ),
    )
    return kfn(dst_all, w_all)


def _dinv_body(d0_ref, d1_ref, o_ref):
    d = d0_ref[...] + d1_ref[...]
    o_ref[...] = jnp.where(d > 0, lax.rsqrt(jnp.maximum(d, 1e-12)), 0.0)


def _dinv_call(deg_parts):
    d0 = deg_parts[0].reshape(N_PAD // D, D)
    d1 = deg_parts[1].reshape(N_PAD // D, D)
    out = pl.pallas_call(
        _dinv_body,
        out_shape=jax.ShapeDtypeStruct((N_PAD // D, D), jnp.float32),
    )(d0, d1)
    return out.reshape(N_PAD)


def _norm_body(t_tile, src_hbm, dst_hbm, w_hbm, dinv_hbm, x_hbm,
               norm_hbm, src1_hbm, dinv_v, x_v, srcs, dsts, ws,
               norms, src1s, lsem):
    cid, sid, wid = _wid()
    sl2 = pl.ds(wid * t_tile, t_tile)
    loads = [
        pltpu.async_copy(src_hbm.at[sl2], srcs, lsem),
        pltpu.async_copy(dst_hbm.at[sl2], dsts, lsem),
        pltpu.async_copy(w_hbm.at[sl2], ws, lsem),
        pltpu.async_copy(dinv_hbm, dinv_v, lsem),
        pltpu.async_copy(x_hbm, x_v, lsem),
    ]
    for l in loads:
        l.wait()

    def chunk(g, _):
        for j in range(K // LANES):
            sl = pl.ds(g * K + j * LANES, LANES)
            s16 = srcs[sl]
            d16 = dsts[sl]
            nv = plsc.load_gather(dinv_v, [s16]) * ws[sl]
            norms[sl] = nv * plsc.load_gather(dinv_v, [d16])
            src1s[sl] = plsc.load_gather(x_v, [s16])
        return 0

    lax.fori_loop(0, t_tile // K, chunk, 0)
    pltpu.async_copy(norms, norm_hbm.at[sl2], lsem).wait()
    pltpu.async_copy(src1s, src1_hbm.at[sl2], lsem).wait()


def _norm_call(src_all, dst_all, w_all, dinv, x_pad, t_tile):
    e_pad = t_tile * NW
    kfn = pl.kernel(
        functools.partial(_norm_body, t_tile),
        out_type=(
            jax.ShapeDtypeStruct((e_pad,), jnp.float32),
            jax.ShapeDtypeStruct((e_pad,), jnp.int32),
        ),
        mesh=_mesh(),
        compiler_params=pltpu.CompilerParams(needs_layout_passes=False),
        scratch_types=[
            pltpu.VMEM((N_PAD,), jnp.float32),
            pltpu.VMEM((N_PAD,), jnp.int32),
            pltpu.VMEM((t_tile,), jnp.int32),
            pltpu.VMEM((t_tile,), jnp.int32),
            pltpu.VMEM((t_tile,), jnp.float32),
            pltpu.VMEM((t_tile,), jnp.float32),
            pltpu.VMEM((t_tile,), jnp.int32),
            pltpu.SemaphoreType.DMA,
        ],
    )
    return kfn(src_all, dst_all, w_all, dinv, x_pad)


KL = 64


def _layer_body(t_tile, y_hbm, src_hbm, dst_hbm, norm_hbm, out_hbm,
                srcb0, srcb1, srcb2, srcb3, dstb0, dstb1, dstb2, dstb3,
                normb0, normb1, normb2, normb3, rows0, rows1, rows2, rows3,
                zrow, s_sp,
                isem0, isem1, isem2, isem3, gsem0, gsem1, gsem2, gsem3,
                ssem0, ssem1, ssem2, ssem3, zsem):
    cid, sid, wid = _wid()
    g_cnt = t_tile // KL
    base = wid * t_tile
    srcb = (srcb0, srcb1, srcb2, srcb3)
    dstb = (dstb0, dstb1, dstb2, dstb3)
    normb = (normb0, normb1, normb2, normb3)
    rows = (rows0, rows1, rows2, rows3)
    isem = (isem0, isem1, isem2, isem3)
    gsem = (gsem0, gsem1, gsem2, gsem3)
    ssem = (ssem0, ssem1, ssem2, ssem3)

    def ifire(c, s):
        off = base + c * KL
        pltpu.async_copy(src_hbm.at[pl.ds(off, KL)], srcb[s], isem[s])
        pltpu.async_copy(dst_hbm.at[pl.ds(off, KL)], dstb[s], isem[s])
        pltpu.async_copy(norm_hbm.at[pl.ds(off, KL)], normb[s], isem[s])

    def iwait(c, s):
        off = base + c * KL
        pltpu.make_async_copy(src_hbm.at[pl.ds(off, KL)], srcb[s], isem[s]).wait()
        pltpu.make_async_copy(dst_hbm.at[pl.ds(off, KL)], dstb[s], isem[s]).wait()
        pltpu.make_async_copy(norm_hbm.at[pl.ds(off, KL)], normb[s], isem[s]).wait()

    def gfire(s):
        pltpu.async_copy(y_hbm.at[srcb[s]], rows[s], gsem[s])

    def gwait(s):
        pltpu.make_async_copy(y_hbm.at[srcb[s]], rows[s], gsem[s]).wait()

    def scale(s):
        buf = rows[s]
        nb_ref = normb[s]

        def srow(r4, _):
            for u in range(4):
                rr = r4 * 4 + u
                nb = plsc.load_gather(nb_ref, [_full16(rr)])
                for j in range(D // LANES):
                    sl = pl.ds(j * LANES, LANES)
                    buf[rr, sl] = buf[rr, sl] * nb
            return 0

        lax.fori_loop(0, KL // 4, srow, 0)

    def sfire(s):
        pltpu.async_copy(rows[s], s_sp.at[dstb[s]], ssem[s], add=True)

    def swait(s):
        pltpu.make_async_copy(rows[s], s_sp.at[dstb[s]], ssem[s]).wait()

    ifire(0, 0)
    ifire(1, 1)
    iwait(0, 0)
    gfire(0)
    iwait(1, 1)
    gfire(1)

    def zfill(i, _):
        for j in range(D // LANES):
            zrow[i, pl.ds(j * LANES, LANES)] = jnp.zeros((LANES,), jnp.float32)
        return 0

    lax.fori_loop(0, ZR, zfill, 0)
    zcopies = [
        pltpu.async_copy(zrow, s_sp.at[pl.ds(sid * RPT + i * ZR, ZR)], zsem)
        for i in range(RPT // ZR)
    ]
    for c in zcopies:
        c.wait()
    plsc.subcore_barrier()

    def step(c, s):
        s2 = (s + 2) % 4

        @pl.when(c >= 2)
        def _():
            swait(s2)

        @pl.when(c + 2 < g_cnt)
        def _():
            ifire(c + 2, s2)

        gwait(s)
        scale(s)
        sfire(s)

        @pl.when(c + 2 < g_cnt)
        def _():
            iwait(c + 2, s2)
            gfire(s2)

    quads = g_cnt // 4

    def quad(q, _):
        for j in range(4):
            step(4 * q + j, j)
        return 0

    lax.fori_loop(0, quads, quad, 0)
    for c in range(4 * quads, g_cnt):
        step(c, c % 4)
    if g_cnt >= 2:
        swait((g_cnt - 2) % 4)
    swait((g_cnt - 1) % 4)

    plsc.subcore_barrier()
    fcopies = [
        pltpu.async_copy(s_sp.at[pl.ds(sid * RPT + i * ZR, ZR)],
                         out_hbm.at[cid, pl.ds(sid * RPT + i * ZR, ZR)], zsem)
        for i in range(RPT // ZR)
    ]
    for c in fcopies:
        c.wait()


def _layer_call(y, src_ids, dst_all, norm, t_tile):
    kfn = pl.kernel(
        functools.partial(_layer_body, t_tile),
        out_type=jax.ShapeDtypeStruct((NC, N_PAD, D), jnp.float32),
        mesh=_mesh(),
        compiler_params=pltpu.CompilerParams(needs_layout_passes=False),
        scratch_types=(
            [pltpu.VMEM((KL,), jnp.int32) for _ in range(4)]
            + [pltpu.VMEM((KL,), jnp.int32) for _ in range(4)]
            + [pltpu.VMEM((KL,), jnp.float32) for _ in range(4)]
            + [pltpu.VMEM((KL, D), jnp.float32) for _ in range(4)]
            + [
                pltpu.VMEM((ZR, D), jnp.float32),
                pltpu.VMEM_SHARED((N_PAD, D), jnp.float32),
            ]
            + [pltpu.SemaphoreType.DMA for _ in range(13)]
        ),
    )
    return kfn(y, src_ids, dst_all, norm)


def _mm0_body(h_ref, w_ref, o_ref):
    o_ref[...] = jnp.dot(h_ref[...], w_ref[...],
                         preferred_element_type=jnp.float32)


def _mm_body(s0_ref, s1_ref, b_ref, w_ref, o_ref):
    h = s0_ref[...] + s1_ref[...] + b_ref[...]
    o_ref[...] = jnp.dot(h, w_ref[...], preferred_element_type=jnp.float32)


def _fin_body(s0_ref, s1_ref, b_ref, o_ref):
    o_ref[...] = s0_ref[...] + s1_ref[...] + b_ref[...]


_BLK = 256


def _mm0_call(h, w):
    return pl.pallas_call(
        _mm0_body,
        grid=(N_PAD // _BLK,),
        in_specs=[
            pl.BlockSpec((_BLK, D), lambda i: (i, 0)),
            pl.BlockSpec((D, D), lambda i: (0, 0)),
        ],
        out_specs=pl.BlockSpec((_BLK, D), lambda i: (i, 0)),
        out_shape=jax.ShapeDtypeStruct((N_PAD, D), jnp.float32),
    )(h, w)


def _mm_call(s, b, w):
    return pl.pallas_call(
        _mm_body,
        grid=(N_PAD // _BLK,),
        in_specs=[
            pl.BlockSpec((_BLK, D), lambda i: (i, 0)),
            pl.BlockSpec((_BLK, D), lambda i: (i, 0)),
            pl.BlockSpec((1, D), lambda i: (0, 0)),
            pl.BlockSpec((D, D), lambda i: (0, 0)),
        ],
        out_specs=pl.BlockSpec((_BLK, D), lambda i: (i, 0)),
        out_shape=jax.ShapeDtypeStruct((N_PAD, D), jnp.float32),
    )(s[0], s[1], b.reshape(1, D), w)


def _fin_call(s, b):
    return pl.pallas_call(
        _fin_body,
        grid=(N_PAD // _BLK,),
        in_specs=[
            pl.BlockSpec((_BLK, D), lambda i: (i, 0)),
            pl.BlockSpec((_BLK, D), lambda i: (i, 0)),
            pl.BlockSpec((1, D), lambda i: (0, 0)),
        ],
        out_specs=pl.BlockSpec((_BLK, D), lambda i: (i, 0)),
        out_shape=jax.ShapeDtypeStruct((N_PAD, D), jnp.float32),
    )(s[0], s[1], b.reshape(1, D))


def kernel(x, edge_index, edge_weight, emb, W0, b0, W1, b1, W2, b2):
    n = emb.shape[0]
    e = edge_weight.shape[0]
    e_all = e + n
    t_tile = -(-e_all // (NW * K)) * K
    e_pad = t_tile * NW

    loop_idx = jnp.arange(n, dtype=jnp.int32)
    src_all = jnp.concatenate([edge_index[0].astype(jnp.int32), loop_idx])
    dst_all = jnp.concatenate([edge_index[1].astype(jnp.int32), loop_idx])
    w_all = jnp.concatenate([edge_weight, jnp.ones((n,), jnp.float32)])
    src_all = jnp.pad(src_all, (0, e_pad - e_all))
    dst_all = jnp.pad(dst_all, (0, e_pad - e_all))
    w_all = jnp.pad(w_all, (0, e_pad - e_all))
    x_pad = jnp.pad(x.astype(jnp.int32), (0, N_PAD - n))
    emb_pad = jnp.pad(emb, ((0, N_PAD - n), (0, 0)))

    deg_parts = _deg_call(dst_all, w_all, t_tile)
    dinv = _dinv_call(deg_parts)
    norm, src1 = _norm_call(src_all, dst_all, w_all, dinv, x_pad, t_tile)

    y = _mm0_call(emb_pad, W0)
    s = _layer_call(y, src1, dst_all, norm, t_tile)
    y = _mm_call(s, b0, W1)
    s = _layer_call(y, src_all, dst_all, norm, t_tile)
    y = _mm_call(s, b1, W2)
    s = _layer_call(y, src_all, dst_all, norm, t_tile)
    out = _fin_call(s, b2)
    return out[:n]


# speedup vs baseline: 19.0193x; 1.3017x over previous
import functools

import jax
import jax.numpy as jnp
from jax import lax
from jax.experimental import pallas as pl
from jax.experimental.pallas import tpu as pltpu
from jax.experimental.pallas import tpu_sc as plsc

D = 128
LANES = 16
NC = 2
NS = 16
NW = NC * NS
K = 128
N_PAD = 10240
RPT = N_PAD // NS
ZR = 64


def _mesh():
    return plsc.VectorSubcoreMesh(core_axis_name="c", subcore_axis_name="s")


def _wid():
    cid = lax.axis_index("c")
    sid = lax.axis_index("s")
    return cid, sid, sid * NC + cid


def _full16(v):
    return jnp.full((LANES,), v, jnp.int32)


def _deg_body(t_tile, dst_hbm, w_hbm, deg_hbm,
              dstb0, dstb1, dstb2, dstb3, wb0, wb1, wb2, wb3, zv, deg_sp,
              isem0, isem1, isem2, isem3, ssem0, ssem1, ssem2, ssem3):
    cid, sid, wid = _wid()
    g_cnt = t_tile // K
    base = wid * t_tile
    dstb = (dstb0, dstb1, dstb2, dstb3)
    wb = (wb0, wb1, wb2, wb3)
    isem = (isem0, isem1, isem2, isem3)
    ssem = (ssem0, ssem1, ssem2, ssem3)

    def ifire(c, s):
        off = base + c * K
        pltpu.async_copy(dst_hbm.at[pl.ds(off, K)], dstb[s], isem[s])
        pltpu.async_copy(w_hbm.at[pl.ds(off, K)], wb[s], isem[s])

    def iwait(c, s):
        off = base + c * K
        pltpu.make_async_copy(dst_hbm.at[pl.ds(off, K)], dstb[s], isem[s]).wait()
        pltpu.make_async_copy(w_hbm.at[pl.ds(off, K)], wb[s], isem[s]).wait()

    def sfire(s):
        pltpu.async_copy(wb[s], deg_sp.at[dstb[s]], ssem[s], add=True)

    def swait(s):
        pltpu.make_async_copy(wb[s], deg_sp.at[dstb[s]], ssem[s]).wait()

    for c in range(min(2, g_cnt)):
        ifire(c, c)

    def zfill(i, _):
        zv[pl.ds(i * LANES, LANES)] = jnp.zeros((LANES,), jnp.float32)
        return 0

    lax.fori_loop(0, RPT // LANES, zfill, 0)
    pltpu.sync_copy(zv, deg_sp.at[pl.ds(sid * RPT, RPT)])
    plsc.subcore_barrier()

    quads = g_cnt // 4

    def quad(q, _):
        for j in range(4):
            c = 4 * q + j
            s = j
            s2 = (j + 2) % 4
            iwait(c, s)
            sfire(s)

            @pl.when(c >= 2)
            def _():
                swait(s2)

            @pl.when(c + 2 < g_cnt)
            def _():
                ifire(c + 2, s2)

        return 0

    lax.fori_loop(0, quads, quad, 0)
    for c in range(4 * quads, g_cnt):
        s = c % 4
        s2 = (s + 2) % 4
        iwait(c, s)
        sfire(s)
        if c >= 2:
            swait(s2)
        if c + 2 < g_cnt:
            ifire(c + 2, s2)
    if g_cnt >= 2:
        swait((g_cnt - 2) % 4)
    swait((g_cnt - 1) % 4)
    plsc.subcore_barrier()
    sl = pl.ds(sid * RPT, RPT)
    pltpu.sync_copy(deg_sp.at[sl], deg_hbm.at[cid, sl])


def _deg_call(dst_all, w_all, t_tile):
    kfn = pl.kernel(
        functools.partial(_deg_body, t_tile),
        out_type=jax.ShapeDtypeStruct((NC, N_PAD), jnp.float32),
        mesh=_mesh(),
        compiler_params=pltpu.CompilerParams(needs_layout_passes=False),
        scratch_types=(
            [pltpu.VMEM((K,), jnp.int32) for _ in range(4)]
            + [pltpu.VMEM((K,), jnp.float32) for _ in range(4)]
            + [
                pltpu.VMEM((RPT,), jnp.float32),
                pltpu.VMEM_SHARED((N_PAD,), jnp.float32),
            ]
            + [pltpu.SemaphoreType.DMA for _ in range(8)]
        ),
    )
    return kfn(dst_all, w_all)


def _dinv_body(d0_ref, d1_ref, o_ref):
    d = d0_ref[...] + d1_ref[...]
    o_ref[...] = jnp.where(d > 0, lax.rsqrt(jnp.maximum(d, 1e-12)), 0.0)


def _dinv_call(deg_parts):
    d0 = deg_parts[0].reshape(N_PAD // D, D)
    d1 = deg_parts[1].reshape(N_PAD // D, D)
    out = pl.pallas_call(
        _dinv_body,
        out_shape=jax.ShapeDtypeStruct((N_PAD // D, D), jnp.float32),
    )(d0, d1)
    return out.reshape(N_PAD)


def _norm_body(t_tile, src_hbm, dst_hbm, w_hbm, dinv_hbm, x_hbm,
               norm_hbm, src1_hbm, dinv_v, x_v, srcs, dsts, ws,
               norms, src1s, lsem):
    cid, sid, wid = _wid()
    sl2 = pl.ds(wid * t_tile, t_tile)
    loads = [
        pltpu.async_copy(src_hbm.at[sl2], srcs, lsem),
        pltpu.async_copy(dst_hbm.at[sl2], dsts, lsem),
        pltpu.async_copy(w_hbm.at[sl2], ws, lsem),
        pltpu.async_copy(dinv_hbm, dinv_v, lsem),
        pltpu.async_copy(x_hbm, x_v, lsem),
    ]
    for l in loads:
        l.wait()

    def chunk(g, _):
        for j in range(K // LANES):
            sl = pl.ds(g * K + j * LANES, LANES)
            s16 = srcs[sl]
            d16 = dsts[sl]
            nv = plsc.load_gather(dinv_v, [s16]) * ws[sl]
            norms[sl] = nv * plsc.load_gather(dinv_v, [d16])
            src1s[sl] = plsc.load_gather(x_v, [s16])
        return 0

    lax.fori_loop(0, t_tile // K, chunk, 0)
    pltpu.async_copy(norms, norm_hbm.at[sl2], lsem).wait()
    pltpu.async_copy(src1s, src1_hbm.at[sl2], lsem).wait()


def _norm_call(src_all, dst_all, w_all, dinv, x_pad, t_tile):
    e_pad = t_tile * NW
    kfn = pl.kernel(
        functools.partial(_norm_body, t_tile),
        out_type=(
            jax.ShapeDtypeStruct((e_pad,), jnp.float32),
            jax.ShapeDtypeStruct((e_pad,), jnp.int32),
        ),
        mesh=_mesh(),
        compiler_params=pltpu.CompilerParams(needs_layout_passes=False),
        scratch_types=[
            pltpu.VMEM((N_PAD,), jnp.float32),
            pltpu.VMEM((N_PAD,), jnp.int32),
            pltpu.VMEM((t_tile,), jnp.int32),
            pltpu.VMEM((t_tile,), jnp.int32),
            pltpu.VMEM((t_tile,), jnp.float32),
            pltpu.VMEM((t_tile,), jnp.float32),
            pltpu.VMEM((t_tile,), jnp.int32),
            pltpu.SemaphoreType.DMA,
        ],
    )
    return kfn(src_all, dst_all, w_all, dinv, x_pad)


KL = 64


def _layer_body(t_tile, y_hbm, src_hbm, dst_hbm, norm_hbm, out_hbm,
                srcb0, srcb1, srcb2, srcb3, dstb0, dstb1, dstb2, dstb3,
                normb0, normb1, normb2, normb3, rows0, rows1, rows2, rows3,
                zrow, s_sp,
                isem0, isem1, isem2, isem3, gsem0, gsem1, gsem2, gsem3,
                ssem0, ssem1, ssem2, ssem3, zsem):
    cid, sid, wid = _wid()
    g_cnt = t_tile // KL
    base = wid * t_tile
    srcb = (srcb0, srcb1, srcb2, srcb3)
    dstb = (dstb0, dstb1, dstb2, dstb3)
    normb = (normb0, normb1, normb2, normb3)
    rows = (rows0, rows1, rows2, rows3)
    isem = (isem0, isem1, isem2, isem3)
    gsem = (gsem0, gsem1, gsem2, gsem3)
    ssem = (ssem0, ssem1, ssem2, ssem3)

    def ifire(c, s):
        off = base + c * KL
        pltpu.async_copy(src_hbm.at[pl.ds(off, KL)], srcb[s], isem[s])
        pltpu.async_copy(dst_hbm.at[pl.ds(off, KL)], dstb[s], isem[s])
        pltpu.async_copy(norm_hbm.at[pl.ds(off, KL)], normb[s], isem[s])

    def iwait(c, s):
        off = base + c * KL
        pltpu.make_async_copy(src_hbm.at[pl.ds(off, KL)], srcb[s], isem[s]).wait()
        pltpu.make_async_copy(dst_hbm.at[pl.ds(off, KL)], dstb[s], isem[s]).wait()
        pltpu.make_async_copy(norm_hbm.at[pl.ds(off, KL)], normb[s], isem[s]).wait()

    def gfire(s):
        pltpu.async_copy(y_hbm.at[srcb[s]], rows[s], gsem[s])

    def gwait(s):
        pltpu.make_async_copy(y_hbm.at[srcb[s]], rows[s], gsem[s]).wait()

    def scale(s):
        buf = rows[s]
        nb_ref = normb[s]

        def srow(r4, _):
            for u in range(4):
                rr = r4 * 4 + u
                nb = plsc.load_gather(nb_ref, [_full16(rr)])
                for j in range(D // LANES):
                    sl = pl.ds(j * LANES, LANES)
                    buf[rr, sl] = buf[rr, sl] * nb
            return 0

        lax.fori_loop(0, KL // 4, srow, 0)

    def sfire(s):
        pltpu.async_copy(rows[s], s_sp.at[dstb[s]], ssem[s], add=True)

    def swait(s):
        pltpu.make_async_copy(rows[s], s_sp.at[dstb[s]], ssem[s]).wait()

    ifire(0, 0)
    ifire(1, 1)
    iwait(0, 0)
    gfire(0)
    iwait(1, 1)
    gfire(1)

    def zfill(i, _):
        for j in range(D // LANES):
            zrow[i, pl.ds(j * LANES, LANES)] = jnp.zeros((LANES,), jnp.float32)
        return 0

    lax.fori_loop(0, ZR, zfill, 0)
    zcopies = [
        pltpu.async_copy(zrow, s_sp.at[pl.ds(sid * RPT + i * ZR, ZR)], zsem)
        for i in range(RPT // ZR)
    ]
    for c in zcopies:
        c.wait()
    plsc.subcore_barrier()

    def step(c, s):
        s2 = (s + 2) % 4

        @pl.when(c >= 2)
        def _():
            swait(s2)

        @pl.when(c + 2 < g_cnt)
        def _():
            ifire(c + 2, s2)

        gwait(s)
        scale(s)
        sfire(s)

        @pl.when(c + 2 < g_cnt)
        def _():
            iwait(c + 2, s2)
            gfire(s2)

    quads = g_cnt // 4

    def quad(q, _):
        for j in range(4):
            step(4 * q + j, j)
        return 0

    lax.fori_loop(0, quads, quad, 0)
    for c in range(4 * quads, g_cnt):
        step(c, c % 4)
    if g_cnt >= 2:
        swait((g_cnt - 2) % 4)
    swait((g_cnt - 1) % 4)

    plsc.subcore_barrier()
    fcopies = [
        pltpu.async_copy(s_sp.at[pl.ds(sid * RPT + i * ZR, ZR)],
                         out_hbm.at[cid, pl.ds(sid * RPT + i * ZR, ZR)], zsem)
        for i in range(RPT // ZR)
    ]
    for c in fcopies:
        c.wait()


def _layer_call(y, src_ids, dst_all, norm, t_tile):
    kfn = pl.kernel(
        functools.partial(_layer_body, t_tile),
        out_type=jax.ShapeDtypeStruct((NC, N_PAD, D), jnp.float32),
        mesh=_mesh(),
        compiler_params=pltpu.CompilerParams(needs_layout_passes=False),
        scratch_types=(
            [pltpu.VMEM((KL,), jnp.int32) for _ in range(4)]
            + [pltpu.VMEM((KL,), jnp.int32) for _ in range(4)]
            + [pltpu.VMEM((KL,), jnp.float32) for _ in range(4)]
            + [pltpu.VMEM((KL, D), jnp.float32) for _ in range(4)]
            + [
                pltpu.VMEM((ZR, D), jnp.float32),
                pltpu.VMEM_SHARED((N_PAD, D), jnp.float32),
            ]
            + [pltpu.SemaphoreType.DMA for _ in range(13)]
        ),
    )
    return kfn(y, src_ids, dst_all, norm)


def _mm0_body(h_ref, w_ref, o_ref):
    o_ref[...] = jnp.dot(h_ref[...], w_ref[...],
                         preferred_element_type=jnp.float32)


def _mm_body(s0_ref, s1_ref, b_ref, w_ref, o_ref):
    h = s0_ref[...] + s1_ref[...] + b_ref[...]
    o_ref[...] = jnp.dot(h, w_ref[...], preferred_element_type=jnp.float32)


def _fin_body(s0_ref, s1_ref, b_ref, o_ref):
    o_ref[...] = s0_ref[...] + s1_ref[...] + b_ref[...]


_BLK = 256


def _mm0_call(h, w):
    return pl.pallas_call(
        _mm0_body,
        grid=(N_PAD // _BLK,),
        in_specs=[
            pl.BlockSpec((_BLK, D), lambda i: (i, 0)),
            pl.BlockSpec((D, D), lambda i: (0, 0)),
        ],
        out_specs=pl.BlockSpec((_BLK, D), lambda i: (i, 0)),
        out_shape=jax.ShapeDtypeStruct((N_PAD, D), jnp.float32),
    )(h, w)


def _mm_call(s, b, w):
    return pl.pallas_call(
        _mm_body,
        grid=(N_PAD // _BLK,),
        in_specs=[
            pl.BlockSpec((_BLK, D), lambda i: (i, 0)),
            pl.BlockSpec((_BLK, D), lambda i: (i, 0)),
            pl.BlockSpec((1, D), lambda i: (0, 0)),
            pl.BlockSpec((D, D), lambda i: (0, 0)),
        ],
        out_specs=pl.BlockSpec((_BLK, D), lambda i: (i, 0)),
        out_shape=jax.ShapeDtypeStruct((N_PAD, D), jnp.float32),
    )(s[0], s[1], b.reshape(1, D), w)


def _fin_call(s, b):
    return pl.pallas_call(
        _fin_body,
        grid=(N_PAD // _BLK,),
        in_specs=[
            pl.BlockSpec((_BLK, D), lambda i: (i, 0)),
            pl.BlockSpec((_BLK, D), lambda i: (i, 0)),
            pl.BlockSpec((1, D), lambda i: (0, 0)),
        ],
        out_specs=pl.BlockSpec((_BLK, D), lambda i: (i, 0)),
        out_shape=jax.ShapeDtypeStruct((N_PAD, D), jnp.float32),
    )(s[0], s[1], b.reshape(1, D))


def kernel(x, edge_index, edge_weight, emb, W0, b0, W1, b1, W2, b2):
    n = emb.shape[0]
    e = edge_weight.shape[0]
    e_all = e + n
    t_tile = -(-e_all // (NW * K)) * K
    e_pad = t_tile * NW

    loop_idx = jnp.arange(n, dtype=jnp.int32)
    src_all = jnp.concatenate([edge_index[0].astype(jnp.int32), loop_idx])
    dst_all = jnp.concatenate([edge_index[1].astype(jnp.int32), loop_idx])
    w_all = jnp.concatenate([edge_weight, jnp.ones((n,), jnp.float32)])
    pad_n = e_pad - e_all
    spread = (jnp.arange(pad_n, dtype=jnp.int32) * 97) % jnp.int32(n)
    src_all = jnp.concatenate([src_all, spread])
    dst_all = jnp.concatenate([dst_all, spread])
    w_all = jnp.pad(w_all, (0, pad_n))
    x_pad = jnp.pad(x.astype(jnp.int32), (0, N_PAD - n))
    emb_pad = jnp.pad(emb, ((0, N_PAD - n), (0, 0)))

    deg_parts = _deg_call(dst_all, w_all, t_tile)
    dinv = _dinv_call(deg_parts)
    norm, src1 = _norm_call(src_all, dst_all, w_all, dinv, x_pad, t_tile)

    y = _mm0_call(emb_pad, W0)
    s = _layer_call(y, src1, dst_all, norm, t_tile)
    y = _mm_call(s, b0, W1)
    s = _layer_call(y, src_all, dst_all, norm, t_tile)
    y = _mm_call(s, b1, W2)
    s = _layer_call(y, src_all, dst_all, norm, t_tile)
    out = _fin_call(s, b2)
    return out[:n]
